# baseline (speedup 1.0000x reference)
import jax
import jax.numpy as jnp
from jax import lax
from jax.experimental import pallas as pl
from jax.experimental.pallas import tpu as pltpu
from jax.experimental.pallas import tpu_sc as plsc

_NC, _NS, _L = 2, 16, 16
_NW = _NC * _NS

_CORR_EPS2 = 100.0
_EPS = 1e-3
_W_START = 50000.0
_W_MAX = 500000.0
_START_RAMP = 50000
_N_RAMP = 100000

_INT_MAX = 2147483647


def _wid():
    return lax.axis_index("s") * _NC + lax.axis_index("c")


def _face_data_body(curr_hbm, next_hbm, faces_hbm,
                    soa_o, ccx_o, ccy_o, ccz_o,
                    nx_o, ny_o, nz_o,
                    faces_v, sem, *bufs):
    ib = bufs[0:9]
    gcur = bufs[9:18]
    gnext = bufs[18:27]
    (cx_b, cy_b, cz_b, b2_b, ccx_b, ccy_b, ccz_b,
     nx_b, ny_b, nz_b) = bufs[27:37]
    wid = _wid()
    fpw = 8192 // _NW
    base = wid * fpw
    pltpu.sync_copy(faces_hbm.at[pl.ds(base * 3, fpw * 3)], faces_v)
    lanes = lax.iota(jnp.int32, _L)
    for i in range(fpw // _L):
        sl16 = pl.ds(i * _L, _L)
        j3 = (i * _L + lanes) * 3
        v0 = plsc.load_gather(faces_v, [j3]) * 3
        v1 = plsc.load_gather(faces_v, [j3 + 1]) * 3
        v2 = plsc.load_gather(faces_v, [j3 + 2]) * 3
        for k, v in enumerate((v0, v1, v2)):
            for c in range(3):
                ib[k * 3 + c][sl16] = v + c
    descs = []
    for kc in range(9):
        descs.append(pltpu.async_copy(curr_hbm.at[ib[kc]], gcur[kc], sem))
        descs.append(pltpu.async_copy(next_hbm.at[ib[kc]], gnext[kc], sem))
    for d in descs:
        d.wait()
    for i in range(fpw // _L):
        sl = pl.ds(i * _L, _L)
        ax = gcur[0][sl]
        ay = gcur[1][sl]
        az = gcur[2][sl]
        bx = gcur[3][sl]
        by = gcur[4][sl]
        bz = gcur[5][sl]
        cx = gcur[6][sl]
        cy = gcur[7][sl]
        cz = gcur[8][sl]
        mx = (ax + bx + cx) / 3.0
        my = (ay + by + cy) / 3.0
        mz = (az + bz + cz) / 3.0
        cx_b[sl] = mx * -2.0
        cy_b[sl] = my * -2.0
        cz_b[sl] = mz * -2.0
        b2_b[sl] = mx * mx + my * my + mz * mz
        ax = gnext[0][sl]
        ay = gnext[1][sl]
        az = gnext[2][sl]
        bx = gnext[3][sl]
        by = gnext[4][sl]
        bz = gnext[5][sl]
        cx = gnext[6][sl]
        cy = gnext[7][sl]
        cz = gnext[8][sl]
        ccx_b[sl] = (ax + bx + cx) / 3.0
        ccy_b[sl] = (ay + by + cy) / 3.0
        ccz_b[sl] = (az + bz + cz) / 3.0
        e1x, e1y, e1z = bx - ax, by - ay, bz - az
        e2x, e2y, e2z = cx - ax, cy - ay, cz - az
        nx_b[sl] = e1y * e2z - e1z * e2y
        ny_b[sl] = e1z * e2x - e1x * e2z
        nz_b[sl] = e1x * e2y - e1y * e2x
    odst = pl.ds(base, fpw)
    pltpu.sync_copy(cx_b, soa_o.at[0, odst])
    pltpu.sync_copy(cy_b, soa_o.at[1, odst])
    pltpu.sync_copy(cz_b, soa_o.at[2, odst])
    pltpu.sync_copy(b2_b, soa_o.at[3, odst])
    pltpu.sync_copy(ccx_b, ccx_o.at[odst])
    pltpu.sync_copy(ccy_b, ccy_o.at[odst])
    pltpu.sync_copy(ccz_b, ccz_o.at[odst])
    pltpu.sync_copy(nx_b, nx_o.at[odst])
    pltpu.sync_copy(ny_b, ny_o.at[odst])
    pltpu.sync_copy(nz_b, nz_o.at[odst])


def _argmin_body(cloth_ref, soa_ref, fidx_ref, md2_ref):
    a = cloth_ref[...]
    ax = a[:, 0:1]
    ay = a[:, 1:2]
    az = a[:, 2:3]
    a2 = ax * ax + ay * ay + az * az
    nchunk = 4
    cw = 8192 // nchunk
    vmin = None
    vch = None
    for j in range(nchunk):
        sl = pl.ds(j * cw, cw)
        b = soa_ref[0:3, sl]
        b2 = soa_ref[3:4, sl]
        t = jnp.dot(a, b, preferred_element_type=jnp.float32)
        s = (a2 + b2) + t
        if j == 0:
            vmin = s
            vch = jnp.zeros(s.shape, jnp.int32)
        else:
            upd = s < vmin
            vmin = jnp.where(upd, s, vmin)
            vch = jnp.where(upd, jnp.full(s.shape, j, jnp.int32), vch)
    rowmin = jnp.min(vmin, axis=1, keepdims=True)
    ids = vch * cw + lax.broadcasted_iota(jnp.int32, vmin.shape, 1)
    cand = jnp.where(vmin == rowmin, ids, jnp.full(vmin.shape, _INT_MAX, jnp.int32))
    fidx_ref[...] = jnp.min(cand, axis=1, keepdims=True)
    md2_ref[...] = rowmin


def _loss_body(pred_hbm, fidx_hbm, md2_hbm,
               ccx_hbm, ccy_hbm, ccz_hbm, nx_hbm, ny_hbm, nz_hbm,
               out_hbm,
               pred_v, fidx_v, md2_v, sem, acc_b, *g):
    wid = _wid()
    ppw = 8192 // _NW
    base = wid * ppw
    pltpu.sync_copy(pred_hbm.at[pl.ds(base * 3, ppw * 3)], pred_v)
    pltpu.sync_copy(fidx_hbm.at[pl.ds(base, ppw)], fidx_v)
    pltpu.sync_copy(md2_hbm.at[pl.ds(base, ppw)], md2_v)
    descs = [
        pltpu.async_copy(ccx_hbm.at[fidx_v], g[0], sem),
        pltpu.async_copy(ccy_hbm.at[fidx_v], g[1], sem),
        pltpu.async_copy(ccz_hbm.at[fidx_v], g[2], sem),
        pltpu.async_copy(nx_hbm.at[fidx_v], g[3], sem),
        pltpu.async_copy(ny_hbm.at[fidx_v], g[4], sem),
        pltpu.async_copy(nz_hbm.at[fidx_v], g[5], sem),
    ]
    for d in descs:
        d.wait()
    lanes = lax.iota(jnp.int32, _L)
    acc = jnp.zeros((_L,), jnp.float32)
    for i in range(ppw // _L):
        p3 = (i * _L + lanes) * 3
        px = plsc.load_gather(pred_v, [p3])
        py = plsc.load_gather(pred_v, [p3 + 1])
        pz = plsc.load_gather(pred_v, [p3 + 2])
        sl = pl.ds(i * _L, _L)
        md = md2_v[sl]
        cx = g[0][sl]
        cy = g[1][sl]
        cz = g[2][sl]
        nx = g[3][sl]
        ny = g[4][sl]
        nz = g[5][sl]
        dot = (px - cx) * nx + (py - cy) * ny + (pz - cz) * nz
        nn2 = nx * nx + ny * ny + nz * nz
        y = plsc.bitcast(jnp.int32(0x5F3759DF) - (plsc.bitcast(nn2, jnp.int32) >> 1),
                         jnp.float32)
        y = y * (1.5 - 0.5 * nn2 * y * y)
        y = y * (1.5 - 0.5 * nn2 * y * y)
        y = y * (1.5 - 0.5 * nn2 * y * y)
        s = nn2 * y
        dist = dot / (s + 1e-12)
        t = jnp.maximum(_EPS - dist, 0.0)
        mask = jnp.where(md < _CORR_EPS2, 1.0, 0.0)
        acc = acc + t * t * t * mask
    acc_b[...] = acc
    pltpu.sync_copy(acc_b, out_hbm.at[wid])


def kernel(cloth_curr_pos, cloth_pred_pos, obstacle_curr_pos,
           obstacle_next_pos, obstacle_faces, iter_num):
    n = cloth_curr_pos.shape[0]
    f = obstacle_faces.shape[0]

    mesh = plsc.VectorSubcoreMesh(core_axis_name="c", subcore_axis_name="s")
    fpw = f // _NW
    f32 = jnp.float32

    sc_params = pltpu.CompilerParams(needs_layout_passes=False)
    face_data = pl.kernel(
        _face_data_body,
        out_type=(jax.ShapeDtypeStruct((4, f), f32),)
        + tuple(jax.ShapeDtypeStruct((f,), f32) for _ in range(6)),
        mesh=mesh,
        compiler_params=sc_params,
        scratch_types=(
            pltpu.VMEM((3 * fpw,), jnp.int32),
            pltpu.SemaphoreType.DMA,
        )
        + tuple(pltpu.VMEM((fpw,), jnp.int32) for _ in range(9))
        + tuple(pltpu.VMEM((fpw,), f32) for _ in range(28)),
    )
    soa, ccx, ccy, ccz, nx, ny, nz = face_data(
        obstacle_curr_pos.reshape(-1),
        obstacle_next_pos.reshape(-1),
        obstacle_faces.reshape(-1),
    )

    blk = 256
    fidx, md2 = pl.pallas_call(
        _argmin_body,
        grid=(n // blk,),
        compiler_params=pltpu.CompilerParams(
            dimension_semantics=("parallel",)),
        in_specs=[
            pl.BlockSpec((blk, 3), lambda i: (i, 0)),
            pl.BlockSpec((4, f), lambda i: (0, 0)),
        ],
        out_specs=[
            pl.BlockSpec((blk, 1), lambda i: (i, 0)),
            pl.BlockSpec((blk, 1), lambda i: (i, 0)),
        ],
        out_shape=[
            jax.ShapeDtypeStruct((n, 1), jnp.int32),
            jax.ShapeDtypeStruct((n, 1), f32),
        ],
    )(cloth_curr_pos, soa)

    ppw = n // _NW
    loss_fn = pl.kernel(
        _loss_body,
        out_type=jax.ShapeDtypeStruct((_NW, _L), f32),
        mesh=mesh,
        compiler_params=sc_params,
        scratch_types=(
            pltpu.VMEM((3 * ppw,), f32),
            pltpu.VMEM((ppw,), jnp.int32),
            pltpu.VMEM((ppw,), f32),
            pltpu.SemaphoreType.DMA,
            pltpu.VMEM((_L,), f32),
        ) + tuple(pltpu.VMEM((ppw,), f32) for _ in range(6)),
    )
    partials = loss_fn(
        cloth_pred_pos.reshape(-1),
        fidx.reshape(-1),
        md2.reshape(-1),
        ccx, ccy, ccz, nx, ny, nz,
    )

    it = jnp.maximum(iter_num - _START_RAMP, 0)
    progress = jnp.minimum(it / _N_RAMP, 1.0)
    weight = (_W_START + (_W_MAX - _W_START) * progress).astype(f32)
    return jnp.sum(partials) * weight


# speedup vs baseline: 1.0312x; 1.0312x over previous
import jax
import jax.numpy as jnp
from jax import lax
from jax.experimental import pallas as pl
from jax.experimental.pallas import tpu as pltpu
from jax.experimental.pallas import tpu_sc as plsc

_NC, _NS, _L = 2, 16, 16
_NW = _NC * _NS

_CORR_EPS2 = 100.0
_EPS = 1e-3
_W_START = 50000.0
_W_MAX = 500000.0
_START_RAMP = 50000
_N_RAMP = 100000

_INT_MAX = 2147483647


def _wid():
    return lax.axis_index("s") * _NC + lax.axis_index("c")


def _face_data_body(curr_hbm, next_hbm, faces_hbm,
                    soa_o, ccx_o, ccy_o, ccz_o,
                    nx_o, ny_o, nz_o,
                    faces_v, sem, *bufs):
    ib = bufs[0:9]
    gcur = bufs[9:18]
    gnext = bufs[18:27]
    (cx_b, cy_b, cz_b, b2_b, ccx_b, ccy_b, ccz_b,
     nx_b, ny_b, nz_b) = bufs[27:37]
    wid = _wid()
    fpw = 8192 // _NW
    base = wid * fpw
    pltpu.sync_copy(faces_hbm.at[pl.ds(base * 3, fpw * 3)], faces_v)
    lanes = lax.iota(jnp.int32, _L)
    for i in range(fpw // _L):
        sl16 = pl.ds(i * _L, _L)
        j3 = (i * _L + lanes) * 3
        v0 = plsc.load_gather(faces_v, [j3]) * 3
        v1 = plsc.load_gather(faces_v, [j3 + 1]) * 3
        v2 = plsc.load_gather(faces_v, [j3 + 2]) * 3
        for k, v in enumerate((v0, v1, v2)):
            for c in range(3):
                ib[k * 3 + c][sl16] = v + c
    descs = []
    for kc in range(9):
        descs.append(pltpu.async_copy(curr_hbm.at[ib[kc]], gcur[kc], sem))
        descs.append(pltpu.async_copy(next_hbm.at[ib[kc]], gnext[kc], sem))
    for d in descs:
        d.wait()
    for i in range(fpw // _L):
        sl = pl.ds(i * _L, _L)
        ax = gcur[0][sl]
        ay = gcur[1][sl]
        az = gcur[2][sl]
        bx = gcur[3][sl]
        by = gcur[4][sl]
        bz = gcur[5][sl]
        cx = gcur[6][sl]
        cy = gcur[7][sl]
        cz = gcur[8][sl]
        mx = (ax + bx + cx) / 3.0
        my = (ay + by + cy) / 3.0
        mz = (az + bz + cz) / 3.0
        cx_b[sl] = mx * -2.0
        cy_b[sl] = my * -2.0
        cz_b[sl] = mz * -2.0
        b2_b[sl] = mx * mx + my * my + mz * mz
        ax = gnext[0][sl]
        ay = gnext[1][sl]
        az = gnext[2][sl]
        bx = gnext[3][sl]
        by = gnext[4][sl]
        bz = gnext[5][sl]
        cx = gnext[6][sl]
        cy = gnext[7][sl]
        cz = gnext[8][sl]
        ccx_b[sl] = (ax + bx + cx) / 3.0
        ccy_b[sl] = (ay + by + cy) / 3.0
        ccz_b[sl] = (az + bz + cz) / 3.0
        e1x, e1y, e1z = bx - ax, by - ay, bz - az
        e2x, e2y, e2z = cx - ax, cy - ay, cz - az
        nx_b[sl] = e1y * e2z - e1z * e2y
        ny_b[sl] = e1z * e2x - e1x * e2z
        nz_b[sl] = e1x * e2y - e1y * e2x
    odst = pl.ds(base, fpw)
    pltpu.sync_copy(cx_b, soa_o.at[0, odst])
    pltpu.sync_copy(cy_b, soa_o.at[1, odst])
    pltpu.sync_copy(cz_b, soa_o.at[2, odst])
    pltpu.sync_copy(b2_b, soa_o.at[3, odst])
    pltpu.sync_copy(ccx_b, ccx_o.at[odst])
    pltpu.sync_copy(ccy_b, ccy_o.at[odst])
    pltpu.sync_copy(ccz_b, ccz_o.at[odst])
    pltpu.sync_copy(nx_b, nx_o.at[odst])
    pltpu.sync_copy(ny_b, ny_o.at[odst])
    pltpu.sync_copy(nz_b, nz_o.at[odst])


def _argmin_body(cloth_ref, soa_ref, fidx_ref, md2_ref):
    a = cloth_ref[...]
    ax = a[:, 0:1]
    ay = a[:, 1:2]
    az = a[:, 2:3]
    a2 = ax * ax + ay * ay + az * az
    nchunk = 8
    cw = 8192 // nchunk
    vmin = None
    vch = None
    for j in range(nchunk):
        sl = pl.ds(j * cw, cw)
        b = soa_ref[0:3, sl]
        b2 = soa_ref[3:4, sl]
        t = jnp.dot(a, b, preferred_element_type=jnp.float32)
        s = (a2 + b2) + t
        if j == 0:
            vmin = s
            vch = jnp.zeros(s.shape, jnp.int32)
        else:
            upd = s < vmin
            vmin = jnp.where(upd, s, vmin)
            vch = jnp.where(upd, jnp.full(s.shape, j, jnp.int32), vch)
    rowmin = jnp.min(vmin, axis=1, keepdims=True)
    ids = vch * cw + lax.broadcasted_iota(jnp.int32, vmin.shape, 1)
    cand = jnp.where(vmin == rowmin, ids, jnp.full(vmin.shape, _INT_MAX, jnp.int32))
    fidx_ref[...] = jnp.min(cand, axis=1, keepdims=True)
    md2_ref[...] = rowmin


def _loss_body(pred_hbm, fidx_hbm, md2_hbm,
               ccx_hbm, ccy_hbm, ccz_hbm, nx_hbm, ny_hbm, nz_hbm,
               out_hbm,
               pred_v, fidx_v, md2_v, sem, acc_b, *g):
    wid = _wid()
    ppw = 8192 // _NW
    base = wid * ppw
    pltpu.sync_copy(pred_hbm.at[pl.ds(base * 3, ppw * 3)], pred_v)
    pltpu.sync_copy(fidx_hbm.at[pl.ds(base, ppw)], fidx_v)
    pltpu.sync_copy(md2_hbm.at[pl.ds(base, ppw)], md2_v)
    descs = [
        pltpu.async_copy(ccx_hbm.at[fidx_v], g[0], sem),
        pltpu.async_copy(ccy_hbm.at[fidx_v], g[1], sem),
        pltpu.async_copy(ccz_hbm.at[fidx_v], g[2], sem),
        pltpu.async_copy(nx_hbm.at[fidx_v], g[3], sem),
        pltpu.async_copy(ny_hbm.at[fidx_v], g[4], sem),
        pltpu.async_copy(nz_hbm.at[fidx_v], g[5], sem),
    ]
    for d in descs:
        d.wait()
    lanes = lax.iota(jnp.int32, _L)
    acc = jnp.zeros((_L,), jnp.float32)
    for i in range(ppw // _L):
        p3 = (i * _L + lanes) * 3
        px = plsc.load_gather(pred_v, [p3])
        py = plsc.load_gather(pred_v, [p3 + 1])
        pz = plsc.load_gather(pred_v, [p3 + 2])
        sl = pl.ds(i * _L, _L)
        md = md2_v[sl]
        cx = g[0][sl]
        cy = g[1][sl]
        cz = g[2][sl]
        nx = g[3][sl]
        ny = g[4][sl]
        nz = g[5][sl]
        dot = (px - cx) * nx + (py - cy) * ny + (pz - cz) * nz
        nn2 = nx * nx + ny * ny + nz * nz
        y = plsc.bitcast(jnp.int32(0x5F3759DF) - (plsc.bitcast(nn2, jnp.int32) >> 1),
                         jnp.float32)
        y = y * (1.5 - 0.5 * nn2 * y * y)
        y = y * (1.5 - 0.5 * nn2 * y * y)
        y = y * (1.5 - 0.5 * nn2 * y * y)
        s = nn2 * y
        dist = dot / (s + 1e-12)
        t = jnp.maximum(_EPS - dist, 0.0)
        mask = jnp.where(md < _CORR_EPS2, 1.0, 0.0)
        acc = acc + t * t * t * mask
    acc_b[...] = acc
    pltpu.sync_copy(acc_b, out_hbm.at[wid])


def kernel(cloth_curr_pos, cloth_pred_pos, obstacle_curr_pos,
           obstacle_next_pos, obstacle_faces, iter_num):
    n = cloth_curr_pos.shape[0]
    f = obstacle_faces.shape[0]

    mesh = plsc.VectorSubcoreMesh(core_axis_name="c", subcore_axis_name="s")
    fpw = f // _NW
    f32 = jnp.float32

    sc_params = pltpu.CompilerParams(needs_layout_passes=False)
    face_data = pl.kernel(
        _face_data_body,
        out_type=(jax.ShapeDtypeStruct((4, f), f32),)
        + tuple(jax.ShapeDtypeStruct((f,), f32) for _ in range(6)),
        mesh=mesh,
        compiler_params=sc_params,
        scratch_types=(
            pltpu.VMEM((3 * fpw,), jnp.int32),
            pltpu.SemaphoreType.DMA,
        )
        + tuple(pltpu.VMEM((fpw,), jnp.int32) for _ in range(9))
        + tuple(pltpu.VMEM((fpw,), f32) for _ in range(28)),
    )
    soa, ccx, ccy, ccz, nx, ny, nz = face_data(
        obstacle_curr_pos.reshape(-1),
        obstacle_next_pos.reshape(-1),
        obstacle_faces.reshape(-1),
    )

    blk = 512
    fidx, md2 = pl.pallas_call(
        _argmin_body,
        grid=(n // blk,),
        compiler_params=pltpu.CompilerParams(
            dimension_semantics=("parallel",)),
        in_specs=[
            pl.BlockSpec((blk, 3), lambda i: (i, 0)),
            pl.BlockSpec((4, f), lambda i: (0, 0)),
        ],
        out_specs=[
            pl.BlockSpec((blk, 1), lambda i: (i, 0)),
            pl.BlockSpec((blk, 1), lambda i: (i, 0)),
        ],
        out_shape=[
            jax.ShapeDtypeStruct((n, 1), jnp.int32),
            jax.ShapeDtypeStruct((n, 1), f32),
        ],
    )(cloth_curr_pos, soa)

    ppw = n // _NW
    loss_fn = pl.kernel(
        _loss_body,
        out_type=jax.ShapeDtypeStruct((_NW, _L), f32),
        mesh=mesh,
        compiler_params=sc_params,
        scratch_types=(
            pltpu.VMEM((3 * ppw,), f32),
            pltpu.VMEM((ppw,), jnp.int32),
            pltpu.VMEM((ppw,), f32),
            pltpu.SemaphoreType.DMA,
            pltpu.VMEM((_L,), f32),
        ) + tuple(pltpu.VMEM((ppw,), f32) for _ in range(6)),
    )
    partials = loss_fn(
        cloth_pred_pos.reshape(-1),
        fidx.reshape(-1),
        md2.reshape(-1),
        ccx, ccy, ccz, nx, ny, nz,
    )

    it = jnp.maximum(iter_num - _START_RAMP, 0)
    progress = jnp.minimum(it / _N_RAMP, 1.0)
    weight = (_W_START + (_W_MAX - _W_START) * progress).astype(f32)
    return jnp.sum(partials) * weight


# speedup vs baseline: 1.0789x; 1.0463x over previous
import jax
import jax.numpy as jnp
from jax import lax
from jax.experimental import pallas as pl
from jax.experimental.pallas import tpu as pltpu
from jax.experimental.pallas import tpu_sc as plsc

_NC, _NS, _L = 2, 16, 16
_NW = _NC * _NS

_CORR_EPS2 = 100.0
_EPS = 1e-3
_W_START = 50000.0
_W_MAX = 500000.0
_START_RAMP = 50000
_N_RAMP = 100000

_INT_MAX = 2147483647


def _wid():
    return lax.axis_index("s") * _NC + lax.axis_index("c")


def _build_face_indices(faces_hbm, faces_v, ib, base, fpw):
    pltpu.sync_copy(faces_hbm.at[pl.ds(base * 3, fpw * 3)], faces_v)
    lanes = lax.iota(jnp.int32, _L)
    for i in range(fpw // _L):
        sl16 = pl.ds(i * _L, _L)
        j3 = (i * _L + lanes) * 3
        v0 = plsc.load_gather(faces_v, [j3]) * 3
        v1 = plsc.load_gather(faces_v, [j3 + 1]) * 3
        v2 = plsc.load_gather(faces_v, [j3 + 2]) * 3
        for k, v in enumerate((v0, v1, v2)):
            for c in range(3):
                ib[k * 3 + c][sl16] = v + c


def _face_curr_body(curr_hbm, faces_hbm, soa_o, faces_v, sem, *bufs):
    ib = bufs[0:9]
    gcur = bufs[9:18]
    cx_b, cy_b, cz_b, b2_b = bufs[18:22]
    wid = _wid()
    fpw = 8192 // _NW
    base = wid * fpw
    _build_face_indices(faces_hbm, faces_v, ib, base, fpw)
    descs = [pltpu.async_copy(curr_hbm.at[ib[kc]], gcur[kc], sem)
             for kc in range(9)]
    for d in descs:
        d.wait()
    for i in range(fpw // _L):
        sl = pl.ds(i * _L, _L)
        ax = gcur[0][sl]
        ay = gcur[1][sl]
        az = gcur[2][sl]
        bx = gcur[3][sl]
        by = gcur[4][sl]
        bz = gcur[5][sl]
        cx = gcur[6][sl]
        cy = gcur[7][sl]
        cz = gcur[8][sl]
        mx = (ax + bx + cx) / 3.0
        my = (ay + by + cy) / 3.0
        mz = (az + bz + cz) / 3.0
        cx_b[sl] = mx * -2.0
        cy_b[sl] = my * -2.0
        cz_b[sl] = mz * -2.0
        b2_b[sl] = mx * mx + my * my + mz * mz
    odst = pl.ds(base, fpw)
    pltpu.sync_copy(cx_b, soa_o.at[0, odst])
    pltpu.sync_copy(cy_b, soa_o.at[1, odst])
    pltpu.sync_copy(cz_b, soa_o.at[2, odst])
    pltpu.sync_copy(b2_b, soa_o.at[3, odst])


def _face_next_body(next_hbm, faces_hbm,
                    ccx_o, ccy_o, ccz_o, nx_o, ny_o, nz_o,
                    faces_v, sem, *bufs):
    ib = bufs[0:9]
    gnext = bufs[9:18]
    ccx_b, ccy_b, ccz_b, nx_b, ny_b, nz_b = bufs[18:24]
    wid = _wid()
    fpw = 8192 // _NW
    base = wid * fpw
    _build_face_indices(faces_hbm, faces_v, ib, base, fpw)
    descs = [pltpu.async_copy(next_hbm.at[ib[kc]], gnext[kc], sem)
             for kc in range(9)]
    for d in descs:
        d.wait()
    for i in range(fpw // _L):
        sl = pl.ds(i * _L, _L)
        ax = gnext[0][sl]
        ay = gnext[1][sl]
        az = gnext[2][sl]
        bx = gnext[3][sl]
        by = gnext[4][sl]
        bz = gnext[5][sl]
        cx = gnext[6][sl]
        cy = gnext[7][sl]
        cz = gnext[8][sl]
        ccx_b[sl] = (ax + bx + cx) / 3.0
        ccy_b[sl] = (ay + by + cy) / 3.0
        ccz_b[sl] = (az + bz + cz) / 3.0
        e1x, e1y, e1z = bx - ax, by - ay, bz - az
        e2x, e2y, e2z = cx - ax, cy - ay, cz - az
        nx_b[sl] = e1y * e2z - e1z * e2y
        ny_b[sl] = e1z * e2x - e1x * e2z
        nz_b[sl] = e1x * e2y - e1y * e2x
    odst = pl.ds(base, fpw)
    pltpu.sync_copy(ccx_b, ccx_o.at[odst])
    pltpu.sync_copy(ccy_b, ccy_o.at[odst])
    pltpu.sync_copy(ccz_b, ccz_o.at[odst])
    pltpu.sync_copy(nx_b, nx_o.at[odst])
    pltpu.sync_copy(ny_b, ny_o.at[odst])
    pltpu.sync_copy(nz_b, nz_o.at[odst])


def _argmin_body(cloth_ref, soa_ref, fidx_ref, md2_ref):
    a = cloth_ref[...]
    ax = a[:, 0:1]
    ay = a[:, 1:2]
    az = a[:, 2:3]
    a2 = ax * ax + ay * ay + az * az
    nchunk = 8
    cw = 8192 // nchunk
    vmin = None
    vch = None
    for j in range(nchunk):
        sl = pl.ds(j * cw, cw)
        b = soa_ref[0:3, sl]
        b2 = soa_ref[3:4, sl]
        t = jnp.dot(a, b, preferred_element_type=jnp.float32)
        s = (a2 + b2) + t
        if j == 0:
            vmin = s
            vch = jnp.zeros(s.shape, jnp.int32)
        else:
            upd = s < vmin
            vmin = jnp.where(upd, s, vmin)
            vch = jnp.where(upd, jnp.full(s.shape, j, jnp.int32), vch)
    rowmin = jnp.min(vmin, axis=1, keepdims=True)
    ids = vch * cw + lax.broadcasted_iota(jnp.int32, vmin.shape, 1)
    cand = jnp.where(vmin == rowmin, ids, jnp.full(vmin.shape, _INT_MAX, jnp.int32))
    fidx_ref[...] = jnp.min(cand, axis=1, keepdims=True)
    md2_ref[...] = rowmin


def _loss_body(pred_hbm, fidx_hbm, md2_hbm,
               ccx_hbm, ccy_hbm, ccz_hbm, nx_hbm, ny_hbm, nz_hbm,
               out_hbm,
               pred_v, fidx_v, md2_v, sem, acc_b, *g):
    wid = _wid()
    ppw = 8192 // _NW
    base = wid * ppw
    pltpu.sync_copy(pred_hbm.at[pl.ds(base * 3, ppw * 3)], pred_v)
    pltpu.sync_copy(fidx_hbm.at[pl.ds(base, ppw)], fidx_v)
    pltpu.sync_copy(md2_hbm.at[pl.ds(base, ppw)], md2_v)
    descs = [
        pltpu.async_copy(ccx_hbm.at[fidx_v], g[0], sem),
        pltpu.async_copy(ccy_hbm.at[fidx_v], g[1], sem),
        pltpu.async_copy(ccz_hbm.at[fidx_v], g[2], sem),
        pltpu.async_copy(nx_hbm.at[fidx_v], g[3], sem),
        pltpu.async_copy(ny_hbm.at[fidx_v], g[4], sem),
        pltpu.async_copy(nz_hbm.at[fidx_v], g[5], sem),
    ]
    for d in descs:
        d.wait()
    lanes = lax.iota(jnp.int32, _L)
    acc = jnp.zeros((_L,), jnp.float32)
    for i in range(ppw // _L):
        p3 = (i * _L + lanes) * 3
        px = plsc.load_gather(pred_v, [p3])
        py = plsc.load_gather(pred_v, [p3 + 1])
        pz = plsc.load_gather(pred_v, [p3 + 2])
        sl = pl.ds(i * _L, _L)
        md = md2_v[sl]
        cx = g[0][sl]
        cy = g[1][sl]
        cz = g[2][sl]
        nx = g[3][sl]
        ny = g[4][sl]
        nz = g[5][sl]
        dot = (px - cx) * nx + (py - cy) * ny + (pz - cz) * nz
        nn2 = nx * nx + ny * ny + nz * nz
        y = plsc.bitcast(jnp.int32(0x5F3759DF) - (plsc.bitcast(nn2, jnp.int32) >> 1),
                         jnp.float32)
        y = y * (1.5 - 0.5 * nn2 * y * y)
        y = y * (1.5 - 0.5 * nn2 * y * y)
        y = y * (1.5 - 0.5 * nn2 * y * y)
        s = nn2 * y
        dist = dot / (s + 1e-12)
        t = jnp.maximum(_EPS - dist, 0.0)
        mask = jnp.where(md < _CORR_EPS2, 1.0, 0.0)
        acc = acc + t * t * t * mask
    acc_b[...] = acc
    pltpu.sync_copy(acc_b, out_hbm.at[wid])


def kernel(cloth_curr_pos, cloth_pred_pos, obstacle_curr_pos,
           obstacle_next_pos, obstacle_faces, iter_num):
    n = cloth_curr_pos.shape[0]
    f = obstacle_faces.shape[0]

    mesh = plsc.VectorSubcoreMesh(core_axis_name="c", subcore_axis_name="s")
    fpw = f // _NW
    f32 = jnp.float32

    sc_params = pltpu.CompilerParams(needs_layout_passes=False)
    face_curr = pl.kernel(
        _face_curr_body,
        out_type=jax.ShapeDtypeStruct((4, f), f32),
        mesh=mesh,
        compiler_params=sc_params,
        scratch_types=(
            pltpu.VMEM((3 * fpw,), jnp.int32),
            pltpu.SemaphoreType.DMA,
        )
        + tuple(pltpu.VMEM((fpw,), jnp.int32) for _ in range(9))
        + tuple(pltpu.VMEM((fpw,), f32) for _ in range(13)),
    )
    face_next = pl.kernel(
        _face_next_body,
        out_type=tuple(jax.ShapeDtypeStruct((f,), f32) for _ in range(6)),
        mesh=mesh,
        compiler_params=sc_params,
        scratch_types=(
            pltpu.VMEM((3 * fpw,), jnp.int32),
            pltpu.SemaphoreType.DMA,
        )
        + tuple(pltpu.VMEM((fpw,), jnp.int32) for _ in range(9))
        + tuple(pltpu.VMEM((fpw,), f32) for _ in range(15)),
    )
    faces_flat = obstacle_faces.reshape(-1)
    soa = face_curr(obstacle_curr_pos.reshape(-1), faces_flat)
    ccx, ccy, ccz, nx, ny, nz = face_next(
        obstacle_next_pos.reshape(-1), faces_flat)

    blk = 256
    fidx, md2 = pl.pallas_call(
        _argmin_body,
        grid=(n // blk,),
        compiler_params=pltpu.CompilerParams(
            dimension_semantics=("parallel",)),
        in_specs=[
            pl.BlockSpec((blk, 3), lambda i: (i, 0)),
            pl.BlockSpec((4, f), lambda i: (0, 0)),
        ],
        out_specs=[
            pl.BlockSpec((blk, 1), lambda i: (i, 0)),
            pl.BlockSpec((blk, 1), lambda i: (i, 0)),
        ],
        out_shape=[
            jax.ShapeDtypeStruct((n, 1), jnp.int32),
            jax.ShapeDtypeStruct((n, 1), f32),
        ],
    )(cloth_curr_pos, soa)

    ppw = n // _NW
    loss_fn = pl.kernel(
        _loss_body,
        out_type=jax.ShapeDtypeStruct((_NW, _L), f32),
        mesh=mesh,
        compiler_params=sc_params,
        scratch_types=(
            pltpu.VMEM((3 * ppw,), f32),
            pltpu.VMEM((ppw,), jnp.int32),
            pltpu.VMEM((ppw,), f32),
            pltpu.SemaphoreType.DMA,
            pltpu.VMEM((_L,), f32),
        ) + tuple(pltpu.VMEM((ppw,), f32) for _ in range(6)),
    )
    partials = loss_fn(
        cloth_pred_pos.reshape(-1),
        fidx.reshape(-1),
        md2.reshape(-1),
        ccx, ccy, ccz, nx, ny, nz,
    )

    it = jnp.maximum(iter_num - _START_RAMP, 0)
    progress = jnp.minimum(it / _N_RAMP, 1.0)
    weight = (_W_START + (_W_MAX - _W_START) * progress).astype(f32)
    return jnp.sum(partials) * weight
